# Initial kernel scaffold; baseline (speedup 1.0000x reference)
import jax
import jax.numpy as jnp
from jax.experimental import pallas as pl


def kernel(d, c):
    raise NotImplementedError("write your pallas kernel here")


# speedup vs baseline: 2.0876x; 2.0876x over previous
import jax
import jax.numpy as jnp
from jax.experimental import pallas as pl

_N = 4096
_D = 256
_C = 32
_R = 64
_BM = 128
_INF = 3e38


def _metrics_block(dblk_ref, dt_ref, sqc_ref, crow_ref, ccol_ref, lim_ref,
                   vec_ref, num_ref):
    blk = pl.program_id(0)
    f32 = jnp.float32

    dot = jnp.dot(dblk_ref[...], dt_ref[...], preferred_element_type=f32)
    key = sqc_ref[...] - 2.0 * dot
    colids = jax.lax.broadcasted_iota(jnp.int32, (_BM, _N), 1)
    rowids = jax.lax.broadcasted_iota(jnp.int32, (_BM, _N), 0) + blk * _BM
    key = jnp.where(colids == rowids, _INF, key)

    crow = crow_ref[...]
    eqf = (crow == ccol_ref[...]).astype(f32)
    cls_lane = jax.lax.broadcasted_iota(jnp.int32, (_BM, _C), 1).astype(f32)
    ohrow = (crow == cls_lane).astype(f32)
    lim = lim_ref[...]

    def step(t, carry):
        key, cnt, r1, r5, r10, mp, num = carry
        m = jnp.min(key, axis=1, keepdims=True)
        ism = key == m
        idx = jnp.min(jnp.where(ism, colids, jnp.int32(_N)),
                      axis=1, keepdims=True)
        oh = colids == idx
        hit = jnp.max(jnp.where(oh, eqf, 0.0), axis=1, keepdims=True)
        cnt = cnt + hit
        s = jnp.sum(cnt)
        r1 = r1 + jnp.where(t == 0, s, 0.0)
        r5 = r5 + jnp.where(t == 4, s / 5.0, 0.0)
        r10 = r10 + jnp.where(t == 9, s / 10.0, 0.0)
        mp = mp + jnp.sum((cnt / (t + 1).astype(f32)) * hit)
        seg_t = jax.lax.dot_general(hit, ohrow, (((0,), (0,)), ((), ())),
                                    preferred_element_type=f32)
        num = num + jnp.where(t.astype(f32) < lim, seg_t, 0.0)
        key = jnp.where(oh, _INF, key)
        return key, cnt, r1, r5, r10, mp, num

    zero = jnp.float32(0.0)
    init = (key, jnp.zeros((_BM, 1), f32), zero, zero, zero, zero,
            jnp.zeros((1, _C), f32))
    _, _, r1, r5, r10, mp, num = jax.lax.fori_loop(0, _R, step, init)

    lane = jax.lax.broadcasted_iota(jnp.int32, (1, 128), 1)
    vecpart = (jnp.where(lane == 0, r1, 0.0) + jnp.where(lane == 1, r5, 0.0)
               + jnp.where(lane == 2, r10, 0.0)
               + jnp.where(lane == 3, mp, 0.0))

    @pl.when(blk == 0)
    def _init():
        vec_ref[...] = jnp.zeros_like(vec_ref)
        num_ref[...] = jnp.zeros_like(num_ref)

    vec_ref[...] += vecpart
    num_ref[...] += num


def kernel(d, c):
    f32 = jnp.float32
    cf = c.astype(f32)
    dt = d.T
    sqc = jnp.sum(d * d, axis=1)[None, :]
    crow = cf[:, None]
    ccol = cf[None, :]
    counts = jnp.sum(cf[:, None] == jnp.arange(_C, dtype=f32)[None, :],
                     axis=0)
    lim = jnp.minimum(counts, f32(_R))[None, :]

    vec, num = pl.pallas_call(
        _metrics_block,
        grid=(_N // _BM,),
        in_specs=[
            pl.BlockSpec((_BM, _D), lambda i: (i, 0)),
            pl.BlockSpec((_D, _N), lambda i: (0, 0)),
            pl.BlockSpec((1, _N), lambda i: (0, 0)),
            pl.BlockSpec((_BM, 1), lambda i: (i, 0)),
            pl.BlockSpec((1, _N), lambda i: (0, 0)),
            pl.BlockSpec((1, _C), lambda i: (0, 0)),
        ],
        out_specs=[
            pl.BlockSpec((1, 128), lambda i: (0, 0)),
            pl.BlockSpec((1, _C), lambda i: (0, 0)),
        ],
        out_shape=[
            jax.ShapeDtypeStruct((1, 128), f32),
            jax.ShapeDtypeStruct((1, _C), f32),
        ],
    )(d, dt, sqc, crow, ccol, lim)

    n = f32(_N)
    recalls = [vec[0, 0] / n, vec[0, 1] / n, vec[0, 2] / n]
    mapr = vec[0, 3] / (n * _R)
    den = jnp.maximum(counts * lim[0], 1.0)
    r_precision = jnp.mean(num[0] / den)
    return jnp.stack(recalls + [mapr, r_precision])
